# baseline (speedup 1.0000x reference)
import functools

import jax
import jax.numpy as jnp
from jax import lax
from jax.experimental import pallas as pl
from jax.experimental.pallas import tpu as pltpu
from jax.experimental.pallas import tpu_sc as plsc

VOCAB = 100000
EMBED_DIM = 64
BATCH = 4096
SEQ = 50
TOK = BATCH * SEQ
OUT_ROWS = TOK * EMBED_DIM // 128

_info = plsc.get_sparse_core_info()
NC, NS = _info.num_cores, _info.num_subcores
NW = NC * NS
BPW = BATCH // NW
BCH = 8
NCH = BPW // BCH
CHTOK = BCH * SEQ
SROWS = CHTOK * EMBED_DIM // 128
NSLOT = 4

_mesh = plsc.VectorSubcoreMesh(core_axis_name="c", subcore_axis_name="s")


@functools.partial(
    pl.kernel,
    mesh=_mesh,
    compiler_params=pltpu.CompilerParams(use_tc_tiling_on_sc=False),
    out_type=jax.ShapeDtypeStruct((OUT_ROWS, 128), jnp.float32),
    scratch_types=[
        pltpu.VMEM((2, NCH, SROWS), jnp.int32),
        pltpu.VMEM((NSLOT, 2, SROWS, EMBED_DIM), jnp.float32),
        pltpu.SemaphoreType.DMA,
        pltpu.SemaphoreType.DMA,
        pltpu.SemaphoreType.DMA,
        pltpu.SemaphoreType.DMA,
        pltpu.SemaphoreType.DMA,
        pltpu.SemaphoreType.DMA,
        pltpu.SemaphoreType.DMA,
        pltpu.SemaphoreType.DMA,
    ],
)
def _lookup(idx_hbm, table_hbm, out_hbm, idx_v, rows_v,
            g0, g1, g2, g3, s0, s1, s2, s3):
    wid = lax.axis_index("s") * NC + lax.axis_index("c")
    base = wid * BPW
    for par in range(2):
        pltpu.sync_copy(idx_hbm.at[par, wid], idx_v.at[par])
    gsem = (g0, g1, g2, g3)
    ssem = (s0, s1, s2, s3)

    def start_g(j, b):
        for par in range(2):
            pltpu.async_copy(
                table_hbm.at[idx_v.at[par, j]],
                rows_v.at[b, par],
                gsem[b],
            )

    def wait_g(j, b):
        for par in range(2):
            pltpu.make_async_copy(
                table_hbm.at[idx_v.at[par, j]],
                rows_v.at[b, par],
                gsem[b],
            ).wait()

    def _s_refs(j, b, par):
        r0 = (base + j * BCH) * SEQ * EMBED_DIM // 128
        return (
            rows_v.at[b, par],
            out_hbm.at[pl.ds(r0, SROWS), pl.ds(par * EMBED_DIM, EMBED_DIM)],
        )

    def start_s(j, b):
        for par in range(2):
            src, dst = _s_refs(j, b, par)
            pltpu.async_copy(src, dst, ssem[b])

    def wait_s(j, b):
        for par in range(2):
            src, dst = _s_refs(j, b, par)
            pltpu.make_async_copy(src, dst, ssem[b]).wait()

    for b in range(NSLOT):
        start_g(b, b)

    def body(i, carry):
        j0 = NSLOT * i
        for b in range(NSLOT):
            j = j0 + b
            wait_g(j, b)
            start_s(j, b)
        for b in range(NSLOT):
            j = j0 + b
            wait_s(j, b)

            @pl.when(j + NSLOT < NCH)
            def _():
                start_g(j + NSLOT, b)

        return carry

    lax.fori_loop(0, NCH // NSLOT, body, 0)


def kernel(sentence, table):
    idx = sentence.reshape(NW, NCH, SROWS, 2).transpose(3, 0, 1, 2)
    out = _lookup(idx, table)
    return out.reshape(BATCH, SEQ, EMBED_DIM)
